# baseline (speedup 1.0000x reference)
import functools

import jax
import jax.numpy as jnp
import numpy as np
from jax import lax
from jax.experimental import pallas as pl
from jax.experimental.pallas import tpu as pltpu
from jax.experimental.pallas import tpu_sc as plsc

H = 1024
F = 4096
E = 8
TOP_K = 2
TM = 256
HW = H // 2
_HI = np.uint32(0xFFFF0000)


def _pack(rows_f32):
    r = rows_f32.astype(jnp.bfloat16).astype(jnp.float32)
    b = lax.bitcast_convert_type(r, jnp.uint32)
    w = (b[:, :HW] >> 16) | (b[:, HW:] & _HI)
    return lax.bitcast_convert_type(w, jnp.int32)


def _unpack(rows_i32):
    wu = lax.bitcast_convert_type(rows_i32, jnp.uint32)
    lo = lax.bitcast_convert_type(wu << 16, jnp.float32)
    hi = lax.bitcast_convert_type(wu & _HI, jnp.float32)
    return jnp.concatenate([lo, hi], axis=1)


def _router_body(x_ref, wr_ref, a1_ref, a2_ref, w1_ref, w2_ref, xi_ref):
    xb = x_ref[...]
    wr = wr_ref[...]
    logits = lax.dot_general(xb, wr, (((1,), (1,)), ((), ())),
                             preferred_element_type=jnp.float32)
    m = jnp.max(logits, axis=1, keepdims=True)
    ex = jnp.exp(logits - m)
    sm = ex / jnp.sum(ex, axis=1, keepdims=True)
    cols = lax.broadcasted_iota(jnp.int32, sm.shape, 1)
    w1v = jnp.max(sm, axis=1, keepdims=True)
    a1v = jnp.min(jnp.where(sm == w1v, cols, E), axis=1, keepdims=True)
    sm2 = jnp.where(cols == a1v, -1.0, sm)
    w2v = jnp.max(sm2, axis=1, keepdims=True)
    a2v = jnp.min(jnp.where(sm2 == w2v, cols, E), axis=1, keepdims=True)
    a1_ref[...] = a1v
    a2_ref[...] = a2v
    w1_ref[...] = w1v
    w2_ref[...] = w2v
    xi_ref[...] = _pack(xb)


def _router(xf, W_router):
    n = xf.shape[0]
    return pl.pallas_call(
        _router_body,
        out_shape=[
            jax.ShapeDtypeStruct((n, 1), jnp.int32),
            jax.ShapeDtypeStruct((n, 1), jnp.int32),
            jax.ShapeDtypeStruct((n, 1), jnp.float32),
            jax.ShapeDtypeStruct((n, 1), jnp.float32),
            jax.ShapeDtypeStruct((n, HW), jnp.int32),
        ],
    )(xf, W_router)


def _sc_dispatch(table, idx, chunk=64):
    n_idx = idx.shape[0]
    info = plsc.get_sparse_core_info()
    nw = info.num_cores * info.num_subcores
    rows_per = n_idx // nw
    n_ch = rows_per // chunk
    mesh = plsc.VectorSubcoreMesh(core_axis_name="c", subcore_axis_name="s")

    @functools.partial(
        pl.kernel,
        mesh=mesh,
        out_type=jax.ShapeDtypeStruct((n_idx, HW), jnp.int32),
        scratch_types=[
            pltpu.VMEM((rows_per,), jnp.int32),
            pltpu.VMEM((chunk, HW), jnp.int32),
            pltpu.VMEM((chunk, HW), jnp.int32),
            pltpu.SemaphoreType.DMA,
            pltpu.SemaphoreType.DMA,
            pltpu.SemaphoreType.DMA,
            pltpu.SemaphoreType.DMA,
        ],
    )
    def k(tab, idx_hbm, out, idx_v, r0, r1, g0, g1, w0, w1):
        wid = lax.axis_index("s") * info.num_cores + lax.axis_index("c")
        base = wid * rows_per
        pltpu.sync_copy(idx_hbm.at[pl.ds(base, rows_per)], idx_v)
        bufs, gs, ws = (r0, r1), (g0, g1), (w0, w1)
        gcp = [None] * n_ch
        wcp = [None] * n_ch
        gcp[0] = pltpu.async_copy(tab.at[idx_v.at[pl.ds(0, chunk)]],
                                  bufs[0], gs[0])
        for c in range(n_ch):
            b = c % 2
            b2 = (c + 1) % 2
            if c + 1 < n_ch:
                if c >= 1:
                    wcp[c - 1].wait()
                gcp[c + 1] = pltpu.async_copy(
                    tab.at[idx_v.at[pl.ds((c + 1) * chunk, chunk)]],
                    bufs[b2], gs[b2])
            gcp[c].wait()
            wcp[c] = pltpu.async_copy(
                bufs[b], out.at[pl.ds(base + c * chunk, chunk)], ws[b])
        for t in range(max(0, n_ch - 2), n_ch):
            wcp[t].wait()

    return k(table, idx)


def _combine_body(ya_ref, yb_ref, out_ref):
    pa = lax.bitcast_convert_type(ya_ref[...], jnp.uint32)
    pb = lax.bitcast_convert_type(yb_ref[...], jnp.uint32)
    a_lo = lax.bitcast_convert_type(pa << 16, jnp.float32)
    a_hi = lax.bitcast_convert_type(pa & _HI, jnp.float32)
    b_lo = lax.bitcast_convert_type(pb << 16, jnp.float32)
    b_hi = lax.bitcast_convert_type(pb & _HI, jnp.float32)
    out_ref[...] = jnp.concatenate([a_lo + b_lo, a_hi + b_hi], axis=1)


def _combine(yun, n):
    bt = 512
    nb = n // bt
    return pl.pallas_call(
        _combine_body,
        grid=(nb,),
        in_specs=[
            pl.BlockSpec((bt, HW), lambda i: (i, 0)),
            pl.BlockSpec((bt, HW), lambda i, nb=nb: (nb + i, 0)),
        ],
        out_specs=pl.BlockSpec((bt, H), lambda i: (i, 0)),
        out_shape=jax.ShapeDtypeStruct((n, H), jnp.float32),
    )(yun, yun)


def _gemm_body(te_ref, xs_ref, w1_ref, w2_ref, sc_ref, out_ref):
    xb = _unpack(xs_ref[...]).astype(jnp.bfloat16)
    pre = lax.dot_general(xb, w1_ref[0], (((1,), (1,)), ((), ())),
                          preferred_element_type=jnp.float32)
    act = jax.nn.gelu(pre, approximate=True).astype(jnp.bfloat16)
    y = lax.dot_general(act, w2_ref[0], (((1,), (0,)), ((), ())),
                        preferred_element_type=jnp.float32)
    out_ref[...] = _pack(y * sc_ref[0])


def _grouped_gemm(xs, w1c, w2c, tile_expert, scales3, n_tiles):
    grid_spec = pltpu.PrefetchScalarGridSpec(
        num_scalar_prefetch=1,
        grid=(n_tiles,),
        in_specs=[
            pl.BlockSpec((TM, HW), lambda m, te: (m, 0)),
            pl.BlockSpec((1, F, H), lambda m, te: (te[m], 0, 0)),
            pl.BlockSpec((1, F, H), lambda m, te: (te[m], 0, 0)),
            pl.BlockSpec((1, TM, 1), lambda m, te: (m, 0, 0)),
        ],
        out_specs=pl.BlockSpec((TM, HW), lambda m, te: (m, 0)),
    )
    return pl.pallas_call(
        _gemm_body,
        grid_spec=grid_spec,
        out_shape=jax.ShapeDtypeStruct((n_tiles * TM, HW), jnp.int32),
        compiler_params=pltpu.CompilerParams(
            dimension_semantics=("arbitrary",)),
    )(tile_expert, xs, w1c, w2c, scales3)


def kernel(x, W_router, w1, w2):
    in_shape = x.shape
    xf = x.reshape(-1, H)
    n = xf.shape[0]
    a_tot = n * TOP_K
    pt = a_tot + E * TM
    n_tiles = pt // TM

    a1, a2, wv1, wv2, xi = _router(xf, W_router)

    e_flat = jnp.stack([a1[:, 0], a2[:, 0]], axis=1).reshape(-1)
    onehot = (e_flat[:, None] == jnp.arange(E)[None, :]).astype(jnp.int32)
    within = jnp.cumsum(onehot, axis=0) - onehot
    rank = jnp.take_along_axis(within, e_flat[:, None], axis=1)[:, 0]
    counts = jnp.sum(onehot, axis=0)
    padded = ((counts + TM - 1) // TM) * TM
    off_dst = jnp.concatenate([jnp.zeros((1,), jnp.int32),
                               jnp.cumsum(padded)[:-1].astype(jnp.int32)])
    dst_a = off_dst[e_flat] + rank
    slot_token = jnp.zeros((pt,), jnp.int32).at[dst_a].set(
        jnp.arange(a_tot, dtype=jnp.int32) // TOP_K)
    w_flat = jnp.stack([wv1[:, 0], wv2[:, 0]], axis=1).reshape(-1)
    slot_scale = jnp.zeros((pt,), jnp.float32).at[dst_a].set(w_flat)
    bounds = jnp.cumsum(padded)
    tile_expert = jnp.clip(
        jnp.searchsorted(bounds, jnp.arange(n_tiles, dtype=jnp.int32) * TM,
                         side="right").astype(jnp.int32), 0, E - 1)

    half = pt // 2
    ht = n_tiles // 2
    scales3 = slot_scale.reshape(n_tiles, TM, 1)
    w1c = w1.astype(jnp.bfloat16).reshape(E, F, H)
    w2c = w2.astype(jnp.bfloat16).reshape(E, F, H)
    xs1 = _sc_dispatch(xi, slot_token[:half], chunk=80)
    ys1 = _grouped_gemm(xs1, w1c, w2c, tile_expert[:ht], scales3[:ht], ht)
    xs2 = _sc_dispatch(xi, slot_token[half:], chunk=80)
    ys2 = _grouped_gemm(xs2, w1c, w2c, tile_expert[ht:], scales3[ht:], ht)
    ys = jnp.concatenate([ys1, ys2], axis=0)

    yun = _sc_dispatch(ys, jnp.concatenate([dst_a[0::2], dst_a[1::2]]))
    out = _combine(yun, n)
    return out.reshape(in_shape)


# speedup vs baseline: 1.0357x; 1.0357x over previous
import functools

import jax
import jax.numpy as jnp
import numpy as np
from jax import lax
from jax.experimental import pallas as pl
from jax.experimental.pallas import tpu as pltpu
from jax.experimental.pallas import tpu_sc as plsc

H = 1024
F = 4096
E = 8
TOP_K = 2
TM = 256
HW = H // 2
_HI = np.uint32(0xFFFF0000)


def _pack(rows_f32):
    r = rows_f32.astype(jnp.bfloat16).astype(jnp.float32)
    b = lax.bitcast_convert_type(r, jnp.uint32)
    w = (b[:, :HW] >> 16) | (b[:, HW:] & _HI)
    return lax.bitcast_convert_type(w, jnp.int32)


def _unpack(rows_i32):
    wu = lax.bitcast_convert_type(rows_i32, jnp.uint32)
    lo = lax.bitcast_convert_type(wu << 16, jnp.float32)
    hi = lax.bitcast_convert_type(wu & _HI, jnp.float32)
    return jnp.concatenate([lo, hi], axis=1)


def _router_body(x_ref, wr_ref, a1_ref, a2_ref, w1_ref, w2_ref, xi_ref):
    xb = x_ref[...]
    wr = wr_ref[...]
    logits = lax.dot_general(xb, wr, (((1,), (1,)), ((), ())),
                             preferred_element_type=jnp.float32)
    m = jnp.max(logits, axis=1, keepdims=True)
    ex = jnp.exp(logits - m)
    sm = ex / jnp.sum(ex, axis=1, keepdims=True)
    cols = lax.broadcasted_iota(jnp.int32, sm.shape, 1)
    w1v = jnp.max(sm, axis=1, keepdims=True)
    a1v = jnp.min(jnp.where(sm == w1v, cols, E), axis=1, keepdims=True)
    sm2 = jnp.where(cols == a1v, -1.0, sm)
    w2v = jnp.max(sm2, axis=1, keepdims=True)
    a2v = jnp.min(jnp.where(sm2 == w2v, cols, E), axis=1, keepdims=True)
    a1_ref[...] = a1v
    a2_ref[...] = a2v
    w1_ref[...] = w1v
    w2_ref[...] = w2v
    xi_ref[...] = _pack(xb)


def _router(xf, W_router):
    n = xf.shape[0]
    return pl.pallas_call(
        _router_body,
        out_shape=[
            jax.ShapeDtypeStruct((n, 1), jnp.int32),
            jax.ShapeDtypeStruct((n, 1), jnp.int32),
            jax.ShapeDtypeStruct((n, 1), jnp.float32),
            jax.ShapeDtypeStruct((n, 1), jnp.float32),
            jax.ShapeDtypeStruct((n, HW), jnp.int32),
        ],
    )(xf, W_router)


def _sc_dispatch(table, idx, chunk=32, nbuf=4):
    n_idx = idx.shape[0]
    info = plsc.get_sparse_core_info()
    nw = info.num_cores * info.num_subcores
    rows_per = n_idx // nw
    n_ch = rows_per // chunk
    nbuf = min(nbuf, n_ch)
    mesh = plsc.VectorSubcoreMesh(core_axis_name="c", subcore_axis_name="s")

    @functools.partial(
        pl.kernel,
        mesh=mesh,
        out_type=jax.ShapeDtypeStruct((n_idx, HW), jnp.int32),
        scratch_types=(
            [pltpu.VMEM((rows_per,), jnp.int32)]
            + [pltpu.VMEM((chunk, HW), jnp.int32)] * nbuf
            + [pltpu.SemaphoreType.DMA] * (2 * nbuf)
        ),
    )
    def k(tab, idx_hbm, out, idx_v, *bufs_sems):
        bufs = bufs_sems[:nbuf]
        gs = bufs_sems[nbuf:2 * nbuf]
        ws = bufs_sems[2 * nbuf:]
        wid = lax.axis_index("s") * info.num_cores + lax.axis_index("c")
        base = wid * rows_per
        pltpu.sync_copy(idx_hbm.at[pl.ds(base, rows_per)], idx_v)

        def gather(c):
            return pltpu.async_copy(
                tab.at[idx_v.at[pl.ds(c * chunk, chunk)]],
                bufs[c % nbuf], gs[c % nbuf])

        gcp = [None] * n_ch
        wcp = [None] * n_ch
        for c0 in range(nbuf):
            gcp[c0] = gather(c0)
        for c in range(n_ch):
            if c >= 1 and (c - 1 + nbuf) < n_ch:
                wcp[c - 1].wait()
                gcp[c - 1 + nbuf] = gather(c - 1 + nbuf)
            gcp[c].wait()
            wcp[c] = pltpu.async_copy(
                bufs[c % nbuf], out.at[pl.ds(base + c * chunk, chunk)],
                ws[c % nbuf])
        for t in range(max(0, n_ch - nbuf), n_ch):
            wcp[t].wait()

    return k(table, idx)


def _combine_body(ya_ref, yb_ref, out_ref):
    pa = lax.bitcast_convert_type(ya_ref[...], jnp.uint32)
    pb = lax.bitcast_convert_type(yb_ref[...], jnp.uint32)
    a_lo = lax.bitcast_convert_type(pa << 16, jnp.float32)
    a_hi = lax.bitcast_convert_type(pa & _HI, jnp.float32)
    b_lo = lax.bitcast_convert_type(pb << 16, jnp.float32)
    b_hi = lax.bitcast_convert_type(pb & _HI, jnp.float32)
    out_ref[...] = jnp.concatenate([a_lo + b_lo, a_hi + b_hi], axis=1)


def _combine(yun, n):
    bt = 512
    nb = n // bt
    return pl.pallas_call(
        _combine_body,
        grid=(nb,),
        in_specs=[
            pl.BlockSpec((bt, HW), lambda i: (i, 0)),
            pl.BlockSpec((bt, HW), lambda i, nb=nb: (nb + i, 0)),
        ],
        out_specs=pl.BlockSpec((bt, H), lambda i: (i, 0)),
        out_shape=jax.ShapeDtypeStruct((n, H), jnp.float32),
    )(yun, yun)


def _gemm_body(te_ref, xs_ref, w1_ref, w2_ref, sc_ref, out_ref):
    xb = _unpack(xs_ref[...]).astype(jnp.bfloat16)
    pre = lax.dot_general(xb, w1_ref[0], (((1,), (1,)), ((), ())),
                          preferred_element_type=jnp.float32)
    act = jax.nn.gelu(pre, approximate=True).astype(jnp.bfloat16)
    y = lax.dot_general(act, w2_ref[0], (((1,), (0,)), ((), ())),
                        preferred_element_type=jnp.float32)
    out_ref[...] = _pack(y * sc_ref[0])


def _grouped_gemm(xs, w1c, w2c, tile_expert, scales3, n_tiles):
    grid_spec = pltpu.PrefetchScalarGridSpec(
        num_scalar_prefetch=1,
        grid=(n_tiles,),
        in_specs=[
            pl.BlockSpec((TM, HW), lambda m, te: (m, 0)),
            pl.BlockSpec((1, F, H), lambda m, te: (te[m], 0, 0)),
            pl.BlockSpec((1, F, H), lambda m, te: (te[m], 0, 0)),
            pl.BlockSpec((1, TM, 1), lambda m, te: (m, 0, 0)),
        ],
        out_specs=pl.BlockSpec((TM, HW), lambda m, te: (m, 0)),
    )
    return pl.pallas_call(
        _gemm_body,
        grid_spec=grid_spec,
        out_shape=jax.ShapeDtypeStruct((n_tiles * TM, HW), jnp.int32),
        compiler_params=pltpu.CompilerParams(
            dimension_semantics=("arbitrary",)),
    )(tile_expert, xs, w1c, w2c, scales3)


def kernel(x, W_router, w1, w2):
    in_shape = x.shape
    xf = x.reshape(-1, H)
    n = xf.shape[0]
    a_tot = n * TOP_K
    pt = a_tot + E * TM
    n_tiles = pt // TM

    a1, a2, wv1, wv2, xi = _router(xf, W_router)

    e_flat = jnp.stack([a1[:, 0], a2[:, 0]], axis=1).reshape(-1)
    onehot = (e_flat[:, None] == jnp.arange(E)[None, :]).astype(jnp.int32)
    within = jnp.cumsum(onehot, axis=0) - onehot
    rank = jnp.take_along_axis(within, e_flat[:, None], axis=1)[:, 0]
    counts = jnp.sum(onehot, axis=0)
    padded = ((counts + TM - 1) // TM) * TM
    off_dst = jnp.concatenate([jnp.zeros((1,), jnp.int32),
                               jnp.cumsum(padded)[:-1].astype(jnp.int32)])
    dst_a = off_dst[e_flat] + rank
    slot_token = jnp.zeros((pt,), jnp.int32).at[dst_a].set(
        jnp.arange(a_tot, dtype=jnp.int32) // TOP_K)
    w_flat = jnp.stack([wv1[:, 0], wv2[:, 0]], axis=1).reshape(-1)
    slot_scale = jnp.zeros((pt,), jnp.float32).at[dst_a].set(w_flat)
    bounds = jnp.cumsum(padded)
    tile_expert = jnp.clip(
        jnp.searchsorted(bounds, jnp.arange(n_tiles, dtype=jnp.int32) * TM,
                         side="right").astype(jnp.int32), 0, E - 1)

    w1c = w1.astype(jnp.bfloat16).reshape(E, F, H)
    w2c = w2.astype(jnp.bfloat16).reshape(E, F, H)
    xs = _sc_dispatch(xi, slot_token)

    ys = _grouped_gemm(xs, w1c, w2c, tile_expert,
                       slot_scale.reshape(n_tiles, TM, 1), n_tiles)

    yun = _sc_dispatch(ys, jnp.concatenate([dst_a[0::2], dst_a[1::2]]))
    out = _combine(yun, n)
    return out.reshape(in_shape)


# speedup vs baseline: 1.1888x; 1.1478x over previous
import functools

import jax
import jax.numpy as jnp
import numpy as np
from jax import lax
from jax.experimental import pallas as pl
from jax.experimental.pallas import tpu as pltpu
from jax.experimental.pallas import tpu_sc as plsc

H = 1024
F = 4096
E = 8
TOP_K = 2
TM = 256
HW = H // 2
_HI = np.uint32(0xFFFF0000)


def _pack(rows_f32):
    r = rows_f32.astype(jnp.bfloat16).astype(jnp.float32)
    b = lax.bitcast_convert_type(r, jnp.uint32)
    w = (b[:, :HW] >> 16) | (b[:, HW:] & _HI)
    return lax.bitcast_convert_type(w, jnp.int32)


def _unpack(rows_i32):
    wu = lax.bitcast_convert_type(rows_i32, jnp.uint32)
    lo = lax.bitcast_convert_type(wu << 16, jnp.float32)
    hi = lax.bitcast_convert_type(wu & _HI, jnp.float32)
    return jnp.concatenate([lo, hi], axis=1)


def _router_body(x_ref, wr_ref, a1_ref, a2_ref, w1_ref, w2_ref, xi_ref):
    xb = x_ref[...]
    wr = wr_ref[...]
    logits = lax.dot_general(xb, wr, (((1,), (1,)), ((), ())),
                             preferred_element_type=jnp.float32)
    m = jnp.max(logits, axis=1, keepdims=True)
    ex = jnp.exp(logits - m)
    sm = ex / jnp.sum(ex, axis=1, keepdims=True)
    cols = lax.broadcasted_iota(jnp.int32, sm.shape, 1)
    w1v = jnp.max(sm, axis=1, keepdims=True)
    a1v = jnp.min(jnp.where(sm == w1v, cols, E), axis=1, keepdims=True)
    sm2 = jnp.where(cols == a1v, -1.0, sm)
    w2v = jnp.max(sm2, axis=1, keepdims=True)
    a2v = jnp.min(jnp.where(sm2 == w2v, cols, E), axis=1, keepdims=True)
    a1_ref[...] = a1v
    a2_ref[...] = a2v
    w1_ref[...] = w1v
    w2_ref[...] = w2v
    xi_ref[...] = _pack(xb)


def _router(xf, W_router):
    n = xf.shape[0]
    return pl.pallas_call(
        _router_body,
        out_shape=[
            jax.ShapeDtypeStruct((n, 1), jnp.int32),
            jax.ShapeDtypeStruct((n, 1), jnp.int32),
            jax.ShapeDtypeStruct((n, 1), jnp.float32),
            jax.ShapeDtypeStruct((n, 1), jnp.float32),
            jax.ShapeDtypeStruct((n, HW), jnp.int32),
        ],
    )(xf, W_router)


def _sc_dispatch(table, idx, chunk=32, nbuf=4):
    n_idx = idx.shape[0]
    info = plsc.get_sparse_core_info()
    nw = info.num_cores * info.num_subcores
    rows_per = n_idx // nw
    n_ch = rows_per // chunk
    nbuf = min(nbuf, n_ch)
    mesh = plsc.VectorSubcoreMesh(core_axis_name="c", subcore_axis_name="s")

    @functools.partial(
        pl.kernel,
        mesh=mesh,
        out_type=jax.ShapeDtypeStruct((n_idx, HW), jnp.int32),
        scratch_types=(
            [pltpu.VMEM((rows_per,), jnp.int32)]
            + [pltpu.VMEM((chunk, HW), jnp.int32)] * nbuf
            + [pltpu.SemaphoreType.DMA] * (2 * nbuf)
        ),
    )
    def k(tab, idx_hbm, out, idx_v, *bufs_sems):
        bufs = bufs_sems[:nbuf]
        gs = bufs_sems[nbuf:2 * nbuf]
        ws = bufs_sems[2 * nbuf:]
        wid = lax.axis_index("s") * info.num_cores + lax.axis_index("c")
        base = wid * rows_per
        pltpu.sync_copy(idx_hbm.at[pl.ds(base, rows_per)], idx_v)

        def gather(c):
            return pltpu.async_copy(
                tab.at[idx_v.at[pl.ds(c * chunk, chunk)]],
                bufs[c % nbuf], gs[c % nbuf])

        gcp = [None] * n_ch
        wcp = [None] * n_ch
        for c0 in range(nbuf):
            gcp[c0] = gather(c0)
        for c in range(n_ch):
            if c >= 1 and (c - 1 + nbuf) < n_ch:
                wcp[c - 1].wait()
                gcp[c - 1 + nbuf] = gather(c - 1 + nbuf)
            gcp[c].wait()
            wcp[c] = pltpu.async_copy(
                bufs[c % nbuf], out.at[pl.ds(base + c * chunk, chunk)],
                ws[c % nbuf])
        for t in range(max(0, n_ch - nbuf), n_ch):
            wcp[t].wait()

    return k(table, idx)


def _sc_staged_gather(table, idx, npass, chunk=64, nbuf=4):
    n_rows, w = table.shape
    wp = w // npass
    n_idx = idx.shape[0]
    info = plsc.get_sparse_core_info()
    nw = info.num_cores * info.num_subcores
    rows_per = n_idx // nw
    n_ch = rows_per // chunk
    nbuf = min(nbuf, n_ch)
    mesh = plsc.VectorSubcoreMesh(core_axis_name="c", subcore_axis_name="s")

    @functools.partial(
        pl.kernel,
        mesh=mesh,
        out_type=jax.ShapeDtypeStruct((n_idx, w), jnp.int32),
        scratch_types=(
            [pltpu.VMEM((rows_per,), jnp.int32),
             pltpu.VMEM_SHARED((n_rows, wp), jnp.int32)]
            + [pltpu.VMEM((chunk, wp), jnp.int32)] * nbuf
            + [pltpu.SemaphoreType.DMA] * (2 * nbuf)
        ),
    )
    def k(tab, idx_hbm, out, idx_v, shared, *bufs_sems):
        bufs = bufs_sems[:nbuf]
        gs = bufs_sems[nbuf:2 * nbuf]
        ws = bufs_sems[2 * nbuf:]
        sid = lax.axis_index("s")
        wid = sid * info.num_cores + lax.axis_index("c")
        base = wid * rows_per
        pltpu.sync_copy(idx_hbm.at[pl.ds(base, rows_per)], idx_v)
        for p in range(npass):
            @pl.when(sid == 0)
            def _():
                pltpu.sync_copy(tab.at[:, pl.ds(p * wp, wp)], shared)
            plsc.subcore_barrier()

            def gather(c):
                return pltpu.async_copy(
                    shared.at[idx_v.at[pl.ds(c * chunk, chunk)]],
                    bufs[c % nbuf], gs[c % nbuf])

            gcp = [None] * n_ch
            wcp = [None] * n_ch
            for c0 in range(nbuf):
                gcp[c0] = gather(c0)
            for c in range(n_ch):
                if c >= 1 and (c - 1 + nbuf) < n_ch:
                    wcp[c - 1].wait()
                    gcp[c - 1 + nbuf] = gather(c - 1 + nbuf)
                gcp[c].wait()
                wcp[c] = pltpu.async_copy(
                    bufs[c % nbuf],
                    out.at[pl.ds(base + c * chunk, chunk), pl.ds(p * wp, wp)],
                    ws[c % nbuf])
            for t in range(max(0, n_ch - nbuf), n_ch):
                wcp[t].wait()
            plsc.subcore_barrier()

    return k(table, idx)


def _combine_body(ya_ref, yb_ref, out_ref):
    pa = lax.bitcast_convert_type(ya_ref[...], jnp.uint32)
    pb = lax.bitcast_convert_type(yb_ref[...], jnp.uint32)
    a_lo = lax.bitcast_convert_type(pa << 16, jnp.float32)
    a_hi = lax.bitcast_convert_type(pa & _HI, jnp.float32)
    b_lo = lax.bitcast_convert_type(pb << 16, jnp.float32)
    b_hi = lax.bitcast_convert_type(pb & _HI, jnp.float32)
    out_ref[...] = jnp.concatenate([a_lo + b_lo, a_hi + b_hi], axis=1)


def _combine(yun, n):
    bt = 512
    nb = n // bt
    return pl.pallas_call(
        _combine_body,
        grid=(nb,),
        in_specs=[
            pl.BlockSpec((bt, HW), lambda i: (i, 0)),
            pl.BlockSpec((bt, HW), lambda i, nb=nb: (nb + i, 0)),
        ],
        out_specs=pl.BlockSpec((bt, H), lambda i: (i, 0)),
        out_shape=jax.ShapeDtypeStruct((n, H), jnp.float32),
    )(yun, yun)


def _gemm_body(te_ref, xs_ref, w1_ref, w2_ref, sc_ref, out_ref):
    xb = _unpack(xs_ref[...]).astype(jnp.bfloat16)
    pre = lax.dot_general(xb, w1_ref[0], (((1,), (1,)), ((), ())),
                          preferred_element_type=jnp.float32)
    act = jax.nn.gelu(pre, approximate=True).astype(jnp.bfloat16)
    y = lax.dot_general(act, w2_ref[0], (((1,), (0,)), ((), ())),
                        preferred_element_type=jnp.float32)
    out_ref[...] = _pack(y * sc_ref[0])


def _grouped_gemm(xs, w1c, w2c, tile_expert, scales3, n_tiles):
    grid_spec = pltpu.PrefetchScalarGridSpec(
        num_scalar_prefetch=1,
        grid=(n_tiles,),
        in_specs=[
            pl.BlockSpec((TM, HW), lambda m, te: (m, 0)),
            pl.BlockSpec((1, F, H), lambda m, te: (te[m], 0, 0)),
            pl.BlockSpec((1, F, H), lambda m, te: (te[m], 0, 0)),
            pl.BlockSpec((1, TM, 1), lambda m, te: (m, 0, 0)),
        ],
        out_specs=pl.BlockSpec((TM, HW), lambda m, te: (m, 0)),
    )
    return pl.pallas_call(
        _gemm_body,
        grid_spec=grid_spec,
        out_shape=jax.ShapeDtypeStruct((n_tiles * TM, HW), jnp.int32),
        compiler_params=pltpu.CompilerParams(
            dimension_semantics=("arbitrary",)),
    )(tile_expert, xs, w1c, w2c, scales3)


def kernel(x, W_router, w1, w2):
    in_shape = x.shape
    xf = x.reshape(-1, H)
    n = xf.shape[0]
    a_tot = n * TOP_K
    pt = a_tot + E * TM
    n_tiles = pt // TM

    a1, a2, wv1, wv2, xi = _router(xf, W_router)

    e_flat = jnp.stack([a1[:, 0], a2[:, 0]], axis=1).reshape(-1)
    onehot = (e_flat[:, None] == jnp.arange(E)[None, :]).astype(jnp.int32)
    within = jnp.cumsum(onehot, axis=0) - onehot
    rank = jnp.take_along_axis(within, e_flat[:, None], axis=1)[:, 0]
    counts = jnp.sum(onehot, axis=0)
    padded = ((counts + TM - 1) // TM) * TM
    off_dst = jnp.concatenate([jnp.zeros((1,), jnp.int32),
                               jnp.cumsum(padded)[:-1].astype(jnp.int32)])
    dst_a = off_dst[e_flat] + rank
    slot_token = jnp.zeros((pt,), jnp.int32).at[dst_a].set(
        jnp.arange(a_tot, dtype=jnp.int32) // TOP_K)
    w_flat = jnp.stack([wv1[:, 0], wv2[:, 0]], axis=1).reshape(-1)
    slot_scale = jnp.zeros((pt,), jnp.float32).at[dst_a].set(w_flat)
    bounds = jnp.cumsum(padded)
    tile_expert = jnp.clip(
        jnp.searchsorted(bounds, jnp.arange(n_tiles, dtype=jnp.int32) * TM,
                         side="right").astype(jnp.int32), 0, E - 1)

    w1c = w1.astype(jnp.bfloat16).reshape(E, F, H)
    w2c = w2.astype(jnp.bfloat16).reshape(E, F, H)
    xs = _sc_staged_gather(xi, slot_token, npass=4)

    ys = _grouped_gemm(xs, w1c, w2c, tile_expert,
                       slot_scale.reshape(n_tiles, TM, 1), n_tiles)

    yun = _sc_staged_gather(ys, jnp.concatenate([dst_a[0::2], dst_a[1::2]]),
                            npass=4)
    out = _combine(yun, n)
    return out.reshape(in_shape)


# speedup vs baseline: 1.2804x; 1.0770x over previous
import functools

import jax
import jax.numpy as jnp
import numpy as np
from jax import lax
from jax.experimental import pallas as pl
from jax.experimental.pallas import tpu as pltpu
from jax.experimental.pallas import tpu_sc as plsc

H = 1024
F = 4096
E = 8
TOP_K = 2
TM = 256
HW = H // 2
_HI = np.uint32(0xFFFF0000)


def _pack(rows_f32):
    r = rows_f32.astype(jnp.bfloat16).astype(jnp.float32)
    b = lax.bitcast_convert_type(r, jnp.uint32)
    w = (b[:, :HW] >> 16) | (b[:, HW:] & _HI)
    return lax.bitcast_convert_type(w, jnp.int32)


def _unpack(rows_i32):
    wu = lax.bitcast_convert_type(rows_i32, jnp.uint32)
    lo = lax.bitcast_convert_type(wu << 16, jnp.float32)
    hi = lax.bitcast_convert_type(wu & _HI, jnp.float32)
    return jnp.concatenate([lo, hi], axis=1)


def _router_body(x_ref, wr_ref, a1_ref, a2_ref, w1_ref, w2_ref, xi_ref):
    xb = x_ref[...]
    wr = wr_ref[...]
    logits = lax.dot_general(xb, wr, (((1,), (1,)), ((), ())),
                             preferred_element_type=jnp.float32)
    m = jnp.max(logits, axis=1, keepdims=True)
    ex = jnp.exp(logits - m)
    sm = ex / jnp.sum(ex, axis=1, keepdims=True)
    cols = lax.broadcasted_iota(jnp.int32, sm.shape, 1)
    w1v = jnp.max(sm, axis=1, keepdims=True)
    a1v = jnp.min(jnp.where(sm == w1v, cols, E), axis=1, keepdims=True)
    sm2 = jnp.where(cols == a1v, -1.0, sm)
    w2v = jnp.max(sm2, axis=1, keepdims=True)
    a2v = jnp.min(jnp.where(sm2 == w2v, cols, E), axis=1, keepdims=True)
    a1_ref[...] = a1v
    a2_ref[...] = a2v
    w1_ref[...] = w1v
    w2_ref[...] = w2v
    xi_ref[...] = _pack(xb)


def _router(xf, W_router):
    n = xf.shape[0]
    return pl.pallas_call(
        _router_body,
        out_shape=[
            jax.ShapeDtypeStruct((n, 1), jnp.int32),
            jax.ShapeDtypeStruct((n, 1), jnp.int32),
            jax.ShapeDtypeStruct((n, 1), jnp.float32),
            jax.ShapeDtypeStruct((n, 1), jnp.float32),
            jax.ShapeDtypeStruct((n, HW), jnp.int32),
        ],
    )(xf, W_router)


def _sc_dispatch(table, idx, chunk=32, nbuf=4):
    n_idx = idx.shape[0]
    info = plsc.get_sparse_core_info()
    nw = info.num_cores * info.num_subcores
    rows_per = n_idx // nw
    n_ch = rows_per // chunk
    nbuf = min(nbuf, n_ch)
    mesh = plsc.VectorSubcoreMesh(core_axis_name="c", subcore_axis_name="s")

    @functools.partial(
        pl.kernel,
        mesh=mesh,
        out_type=jax.ShapeDtypeStruct((n_idx, HW), jnp.int32),
        scratch_types=(
            [pltpu.VMEM((rows_per,), jnp.int32)]
            + [pltpu.VMEM((chunk, HW), jnp.int32)] * nbuf
            + [pltpu.SemaphoreType.DMA] * (2 * nbuf)
        ),
    )
    def k(tab, idx_hbm, out, idx_v, *bufs_sems):
        bufs = bufs_sems[:nbuf]
        gs = bufs_sems[nbuf:2 * nbuf]
        ws = bufs_sems[2 * nbuf:]
        wid = lax.axis_index("s") * info.num_cores + lax.axis_index("c")
        base = wid * rows_per
        pltpu.sync_copy(idx_hbm.at[pl.ds(base, rows_per)], idx_v)

        def gather(c):
            return pltpu.async_copy(
                tab.at[idx_v.at[pl.ds(c * chunk, chunk)]],
                bufs[c % nbuf], gs[c % nbuf])

        gcp = [None] * n_ch
        wcp = [None] * n_ch
        for c0 in range(nbuf):
            gcp[c0] = gather(c0)
        for c in range(n_ch):
            if c >= 1 and (c - 1 + nbuf) < n_ch:
                wcp[c - 1].wait()
                gcp[c - 1 + nbuf] = gather(c - 1 + nbuf)
            gcp[c].wait()
            wcp[c] = pltpu.async_copy(
                bufs[c % nbuf], out.at[pl.ds(base + c * chunk, chunk)],
                ws[c % nbuf])
        for t in range(max(0, n_ch - nbuf), n_ch):
            wcp[t].wait()

    return k(table, idx)


def _sc_staged_gather(table, idx, npass, chunk=64, nbuf=4):
    n_rows, w = table.shape
    wp = w // npass
    n_idx = idx.shape[0]
    info = plsc.get_sparse_core_info()
    nw = info.num_cores * info.num_subcores
    rows_per = n_idx // nw
    n_ch = rows_per // chunk
    nbuf = min(nbuf, n_ch)
    mesh = plsc.VectorSubcoreMesh(core_axis_name="c", subcore_axis_name="s")

    @functools.partial(
        pl.kernel,
        mesh=mesh,
        out_type=jax.ShapeDtypeStruct((n_idx, w), jnp.int32),
        scratch_types=(
            [pltpu.VMEM((rows_per,), jnp.int32),
             pltpu.VMEM_SHARED((n_rows, wp), jnp.int32)]
            + [pltpu.VMEM((chunk, wp), jnp.int32)] * nbuf
            + [pltpu.SemaphoreType.DMA] * (2 * nbuf)
        ),
    )
    def k(tab, idx_hbm, out, idx_v, shared, *bufs_sems):
        bufs = bufs_sems[:nbuf]
        gs = bufs_sems[nbuf:2 * nbuf]
        ws = bufs_sems[2 * nbuf:]
        sid = lax.axis_index("s")
        wid = sid * info.num_cores + lax.axis_index("c")
        base = wid * rows_per
        pltpu.sync_copy(idx_hbm.at[pl.ds(base, rows_per)], idx_v)
        for p in range(npass):
            @pl.when(sid == 0)
            def _():
                pltpu.sync_copy(tab.at[:, pl.ds(p * wp, wp)], shared)
            plsc.subcore_barrier()

            def gather(c):
                return pltpu.async_copy(
                    shared.at[idx_v.at[pl.ds(c * chunk, chunk)]],
                    bufs[c % nbuf], gs[c % nbuf])

            gcp = [None] * n_ch
            wcp = [None] * n_ch
            for c0 in range(nbuf):
                gcp[c0] = gather(c0)
            for c in range(n_ch):
                if c >= 1 and (c - 1 + nbuf) < n_ch:
                    wcp[c - 1].wait()
                    gcp[c - 1 + nbuf] = gather(c - 1 + nbuf)
                gcp[c].wait()
                wcp[c] = pltpu.async_copy(
                    bufs[c % nbuf],
                    out.at[pl.ds(base + c * chunk, chunk), pl.ds(p * wp, wp)],
                    ws[c % nbuf])
            for t in range(max(0, n_ch - nbuf), n_ch):
                wcp[t].wait()
            plsc.subcore_barrier()

    return k(table, idx)


def _combine_body(ya_ref, yb_ref, out_ref):
    pa = lax.bitcast_convert_type(ya_ref[...], jnp.uint32)
    pb = lax.bitcast_convert_type(yb_ref[...], jnp.uint32)
    a_lo = lax.bitcast_convert_type(pa << 16, jnp.float32)
    a_hi = lax.bitcast_convert_type(pa & _HI, jnp.float32)
    b_lo = lax.bitcast_convert_type(pb << 16, jnp.float32)
    b_hi = lax.bitcast_convert_type(pb & _HI, jnp.float32)
    out_ref[...] = jnp.concatenate([a_lo + b_lo, a_hi + b_hi], axis=1)


def _combine(yun, n):
    bt = 512
    nb = n // bt
    return pl.pallas_call(
        _combine_body,
        grid=(nb,),
        in_specs=[
            pl.BlockSpec((bt, HW), lambda i: (i, 0)),
            pl.BlockSpec((bt, HW), lambda i, nb=nb: (nb + i, 0)),
        ],
        out_specs=pl.BlockSpec((bt, H), lambda i: (i, 0)),
        out_shape=jax.ShapeDtypeStruct((n, H), jnp.float32),
    )(yun, yun)


def _gemm_body(te_ref, xs_ref, w1_ref, w2_ref, sc_ref, out_ref):
    xb = _unpack(xs_ref[...]).astype(jnp.bfloat16)
    pre = lax.dot_general(xb, w1_ref[0], (((1,), (1,)), ((), ())),
                          preferred_element_type=jnp.float32)
    act = jax.nn.gelu(pre, approximate=True).astype(jnp.bfloat16)
    y = lax.dot_general(act, w2_ref[0], (((1,), (0,)), ((), ())),
                        preferred_element_type=jnp.float32)
    out_ref[...] = _pack(y * sc_ref[0])


def _grouped_gemm(xs, w1c, w2c, tile_expert, scales3, n_tiles):
    grid_spec = pltpu.PrefetchScalarGridSpec(
        num_scalar_prefetch=1,
        grid=(n_tiles,),
        in_specs=[
            pl.BlockSpec((TM, HW), lambda m, te: (m, 0)),
            pl.BlockSpec((1, F, H), lambda m, te: (te[m], 0, 0)),
            pl.BlockSpec((1, F, H), lambda m, te: (te[m], 0, 0)),
            pl.BlockSpec((1, TM, 1), lambda m, te: (m, 0, 0)),
        ],
        out_specs=pl.BlockSpec((TM, HW), lambda m, te: (m, 0)),
    )
    return pl.pallas_call(
        _gemm_body,
        grid_spec=grid_spec,
        out_shape=jax.ShapeDtypeStruct((n_tiles * TM, HW), jnp.int32),
        compiler_params=pltpu.CompilerParams(
            dimension_semantics=("arbitrary",)),
    )(tile_expert, xs, w1c, w2c, scales3)


def kernel(x, W_router, w1, w2):
    in_shape = x.shape
    xf = x.reshape(-1, H)
    n = xf.shape[0]
    a_tot = n * TOP_K
    pt = a_tot + E * TM
    n_tiles = pt // TM

    a1, a2, wv1, wv2, xi = _router(xf, W_router)

    e_flat = jnp.stack([a1[:, 0], a2[:, 0]], axis=1).reshape(-1)
    onehot = (e_flat[:, None] == jnp.arange(E)[None, :]).astype(jnp.int32)
    within = jnp.cumsum(onehot, axis=0) - onehot
    rank = jnp.sum(within * onehot, axis=1)
    counts = jnp.sum(onehot, axis=0)
    padded = ((counts + TM - 1) // TM) * TM
    off_dst = jnp.concatenate([jnp.zeros((1,), jnp.int32),
                               jnp.cumsum(padded)[:-1].astype(jnp.int32)])
    dst_a = off_dst[e_flat] + rank
    slot_token = jnp.zeros((pt,), jnp.int32).at[dst_a].set(
        jnp.arange(a_tot, dtype=jnp.int32) // TOP_K)
    w_flat = jnp.stack([wv1[:, 0], wv2[:, 0]], axis=1).reshape(-1)
    slot_scale = jnp.zeros((pt,), jnp.float32).at[dst_a].set(w_flat)
    bounds = jnp.cumsum(padded)
    tile_expert = jnp.clip(
        jnp.sum((bounds[None, :] <= jnp.arange(n_tiles, dtype=jnp.int32)[:, None]
                 * TM).astype(jnp.int32), axis=1), 0, E - 1)

    w1c = w1.astype(jnp.bfloat16).reshape(E, F, H)
    w2c = w2.astype(jnp.bfloat16).reshape(E, F, H)
    xs = _sc_staged_gather(xi, slot_token, npass=4, chunk=80)

    ys = _grouped_gemm(xs, w1c, w2c, tile_expert,
                       slot_scale.reshape(n_tiles, TM, 1), n_tiles)

    yun = _sc_staged_gather(ys, jnp.concatenate([dst_a[0::2], dst_a[1::2]]),
                            npass=4)
    out = _combine(yun, n)
    return out.reshape(in_shape)
